# baseline (speedup 1.0000x reference)
import functools

import jax
import jax.numpy as jnp
from jax import lax
from jax.experimental import pallas as pl
from jax.experimental.pallas import tpu as pltpu
from jax.experimental.pallas import tpu_sc as plsc


def _sc_gather(table, idx, B, V, E):
    info = plsc.get_sparse_core_info()
    NC, NS = 1, info.num_subcores
    NW = NC * NS
    b_per_w = B // NW

    mesh = plsc.VectorSubcoreMesh(
        core_axis_name="c", subcore_axis_name="s", num_cores=NC)

    @functools.partial(
        pl.kernel,
        mesh=mesh,
        out_type=jax.ShapeDtypeStruct((B, E), jnp.float32),
        scratch_types=[
            pltpu.VMEM((b_per_w,), jnp.int32),
            pltpu.VMEM((b_per_w, E), jnp.float32),
            pltpu.SemaphoreType.DMA,
        ],
    )
    def gather_kernel(table_hbm, idx_hbm, out_hbm, idx_v, rows_v, sem):
        wid = lax.axis_index("s") * NC + lax.axis_index("c")
        base = wid * b_per_w
        pltpu.sync_copy(idx_hbm.at[pl.ds(base, b_per_w)], idx_v)
        pltpu.async_copy(table_hbm.at[idx_v], rows_v, sem).wait()
        pltpu.sync_copy(rows_v, out_hbm.at[pl.ds(base, b_per_w)])

    return gather_kernel(table, idx)


def _matmul_body(w_ref, e_ref, b_ref, o_ref, et_ref):
    j = pl.program_id(1)

    @pl.when(j == 0)
    def _():
        et_ref[...] = jnp.transpose(
            e_ref[...].astype(jnp.bfloat16), (1, 0))

    w = w_ref[...].astype(jnp.bfloat16)
    acc = lax.dot_general(
        w, et_ref[...],
        dimension_numbers=(((1,), (0,)), ((), ())),
        preferred_element_type=jnp.float32,
    )
    bcol = jnp.transpose(b_ref[...], (1, 0))
    o_ref[...] = acc + bcol


def kernel(input_seq, token_table, out_weight, out_bias):
    V, E = token_table.shape
    B = input_seq.shape[0]

    idx = input_seq.astype(jnp.int32)
    embeds = jnp.take(token_table, idx, axis=0)

    bn = 5120
    ncore = 2
    inner = pl.cdiv(pl.cdiv(V, bn), ncore)
    grid = (ncore, inner)
    bias2d = out_bias.reshape(1, V)

    out_t = pl.pallas_call(
        _matmul_body,
        grid=grid,
        in_specs=[
            pl.BlockSpec((bn, E), lambda i, j: (i * inner + j, 0)),
            pl.BlockSpec((B, E), lambda i, j: (0, 0)),
            pl.BlockSpec((1, bn), lambda i, j: (0, i * inner + j)),
        ],
        out_specs=pl.BlockSpec((bn, B), lambda i, j: (i * inner + j, 0)),
        out_shape=jax.ShapeDtypeStruct((V, B), jnp.float32),
        scratch_shapes=[pltpu.VMEM((E, B), jnp.bfloat16)],
        compiler_params=pltpu.CompilerParams(
            dimension_semantics=("parallel", "arbitrary"),
        ),
    )(out_weight, embeds, bias2d)
    return out_t.T


# speedup vs baseline: 1.0064x; 1.0064x over previous
import functools

import jax
import jax.numpy as jnp
from jax import lax
from jax.experimental import pallas as pl
from jax.experimental.pallas import tpu as pltpu
from jax.experimental.pallas import tpu_sc as plsc


def _sc_gather(table, idx, B, V, E):
    info = plsc.get_sparse_core_info()
    NC, NS = 1, info.num_subcores
    NW = NC * NS
    b_per_w = B // NW

    mesh = plsc.VectorSubcoreMesh(
        core_axis_name="c", subcore_axis_name="s", num_cores=NC)

    @functools.partial(
        pl.kernel,
        mesh=mesh,
        out_type=jax.ShapeDtypeStruct((B, E), jnp.float32),
        scratch_types=[
            pltpu.VMEM((b_per_w,), jnp.int32),
            pltpu.VMEM((b_per_w, E), jnp.float32),
            pltpu.SemaphoreType.DMA,
        ],
    )
    def gather_kernel(table_hbm, idx_hbm, out_hbm, idx_v, rows_v, sem):
        wid = lax.axis_index("s") * NC + lax.axis_index("c")
        base = wid * b_per_w
        pltpu.sync_copy(idx_hbm.at[pl.ds(base, b_per_w)], idx_v)
        pltpu.async_copy(table_hbm.at[idx_v], rows_v, sem).wait()
        pltpu.sync_copy(rows_v, out_hbm.at[pl.ds(base, b_per_w)])

    return gather_kernel(table, idx)


def _matmul_body(w_ref, e_ref, b_ref, o_ref, et_ref):
    j = pl.program_id(1)

    @pl.when(j == 0)
    def _():
        et_ref[...] = jnp.transpose(
            e_ref[...].astype(jnp.bfloat16), (1, 0))

    w = w_ref[...].astype(jnp.bfloat16)
    acc = lax.dot_general(
        w, et_ref[...],
        dimension_numbers=(((1,), (0,)), ((), ())),
        preferred_element_type=jnp.float32,
    )
    bcol = jnp.transpose(b_ref[...], (1, 0))
    o_ref[...] = acc + bcol


def kernel(input_seq, token_table, out_weight, out_bias):
    V, E = token_table.shape
    B = input_seq.shape[0]

    idx = input_seq.astype(jnp.int32)
    embeds = _sc_gather(token_table, idx, B, V, E)

    bn = 4608
    ncore = 2
    inner = pl.cdiv(pl.cdiv(V, bn), ncore)
    grid = (ncore, inner)
    bias2d = out_bias.reshape(1, V)

    out_t = pl.pallas_call(
        _matmul_body,
        grid=grid,
        in_specs=[
            pl.BlockSpec((bn, E), lambda i, j: (i * inner + j, 0)),
            pl.BlockSpec((B, E), lambda i, j: (0, 0)),
            pl.BlockSpec((1, bn), lambda i, j: (0, i * inner + j)),
        ],
        out_specs=pl.BlockSpec((bn, B), lambda i, j: (i * inner + j, 0)),
        out_shape=jax.ShapeDtypeStruct((V, B), jnp.float32),
        scratch_shapes=[pltpu.VMEM((E, B), jnp.bfloat16)],
        compiler_params=pltpu.CompilerParams(
            dimension_semantics=("parallel", "arbitrary"),
        ),
    )(out_weight, embeds, bias2d)
    return out_t.T


# speedup vs baseline: 1.0208x; 1.0143x over previous
import functools

import jax
import jax.numpy as jnp
from jax import lax
from jax.experimental import pallas as pl
from jax.experimental.pallas import tpu as pltpu
from jax.experimental.pallas import tpu_sc as plsc


def _sc_gather(table, idx, B, V, E):
    info = plsc.get_sparse_core_info()
    NC, NS = 1, info.num_subcores
    NW = NC * NS
    b_per_w = B // NW

    mesh = plsc.VectorSubcoreMesh(
        core_axis_name="c", subcore_axis_name="s", num_cores=NC)

    @functools.partial(
        pl.kernel,
        mesh=mesh,
        out_type=jax.ShapeDtypeStruct((B, E), jnp.float32),
        scratch_types=[
            pltpu.VMEM((b_per_w,), jnp.int32),
            pltpu.VMEM((b_per_w, E), jnp.float32),
            pltpu.SemaphoreType.DMA,
        ],
    )
    def gather_kernel(table_hbm, idx_hbm, out_hbm, idx_v, rows_v, sem):
        wid = lax.axis_index("s") * NC + lax.axis_index("c")
        base = wid * b_per_w
        pltpu.sync_copy(idx_hbm.at[pl.ds(base, b_per_w)], idx_v)
        pltpu.async_copy(table_hbm.at[idx_v], rows_v, sem).wait()
        pltpu.sync_copy(rows_v, out_hbm.at[pl.ds(base, b_per_w)])

    return gather_kernel(table, idx)


def _matmul_body(w_ref, e_ref, b_ref, o_ref, et_ref):
    j = pl.program_id(1)

    @pl.when(j == 0)
    def _():
        et_ref[...] = jnp.transpose(
            e_ref[...].astype(jnp.bfloat16), (1, 0))

    w = w_ref[...].astype(jnp.bfloat16)
    acc = lax.dot_general(
        w, et_ref[...],
        dimension_numbers=(((1,), (0,)), ((), ())),
        preferred_element_type=jnp.float32,
    )
    bcol = jnp.transpose(b_ref[...], (1, 0))
    o_ref[...] = acc + bcol


def kernel(input_seq, token_table, out_weight, out_bias):
    V, E = token_table.shape
    B = input_seq.shape[0]

    idx = input_seq.astype(jnp.int32)
    embeds = _sc_gather(token_table, idx, B, V, E)

    bn = 5120
    ncore = 2
    inner = pl.cdiv(pl.cdiv(V, bn), ncore)
    grid = (ncore, inner)
    bias2d = out_bias.reshape(1, V)

    out_t = pl.pallas_call(
        _matmul_body,
        grid=grid,
        in_specs=[
            pl.BlockSpec((bn, E), lambda i, j: (i * inner + j, 0)),
            pl.BlockSpec((B, E), lambda i, j: (0, 0)),
            pl.BlockSpec((1, bn), lambda i, j: (0, i * inner + j)),
        ],
        out_specs=pl.BlockSpec((bn, B), lambda i, j: (i * inner + j, 0)),
        out_shape=jax.ShapeDtypeStruct((V, B), jnp.float32),
        scratch_shapes=[pltpu.VMEM((E, B), jnp.bfloat16)],
        compiler_params=pltpu.CompilerParams(
            dimension_semantics=("parallel", "arbitrary"),
        ),
    )(out_weight, embeds, bias2d)
    return out_t.T
